# baseline (speedup 1.0000x reference)
import functools

import jax
import jax.numpy as jnp
from jax import lax
from jax.experimental import pallas as pl
from jax.experimental.pallas import tpu as pltpu
from jax.experimental.pallas import tpu_sc as plsc

_L = 128
_NBUF = 2


def _make_gather(b: int, s: int, d: int):
    info = plsc.get_sparse_core_info()
    nc, ns = info.num_cores, info.num_subcores
    nw = nc * ns
    assert b % (nw * _L) == 0
    b_w = b // nw
    jmax = b_w // _L
    chunks_w = s * jmax
    assert chunks_w % _NBUF == 0
    n_groups = chunks_w // _NBUF
    mesh = plsc.VectorSubcoreMesh(core_axis_name="c", subcore_axis_name="s")

    @functools.partial(
        pl.kernel,
        mesh=mesh,
        out_type=jax.ShapeDtypeStruct((s, d, b), jnp.float32),
        scratch_types=[
            pltpu.VMEM((s, b_w), jnp.int32),
            pltpu.VMEM((s, b_w), jnp.int32),
            pltpu.VMEM((_NBUF, _L, _L), jnp.float32),
            pltpu.VMEM((_NBUF, d, _L), jnp.float32),
            [pltpu.SemaphoreType.DMA] * _NBUF,
            [pltpu.SemaphoreType.DMA] * _NBUF,
        ],
        compiler_params=pltpu.CompilerParams(needs_layout_passes=False),
    )
    def gather(ids_hbm, half_hbm, table_hbm, out_hbm, raw_v, half_v,
               bufs, tbufs, gsem, ssem):
        wid = lax.axis_index("s") * nc + lax.axis_index("c")
        b0 = wid * b_w

        pltpu.sync_copy(ids_hbm.at[:, pl.ds(b0, b_w)], raw_v)
        pltpu.sync_copy(half_hbm.at[:, pl.ds(b0, b_w)], half_v)

        def split(k):
            sq = k // jmax
            jj = k % jmax
            return sq, jj

        def g_start(k, bslot):
            sq, jj = split(k)
            pltpu.async_copy(
                table_hbm.at[half_v.at[sq, pl.ds(jj * _L, _L)]],
                bufs.at[bslot], gsem[bslot])

        def g_wait(bslot):
            pltpu.make_async_copy(table_hbm.at[half_v.at[0, pl.ds(0, _L)]],
                                  bufs.at[bslot], gsem[bslot]).wait()

        def merge(k, bslot):
            sq, jj = split(k)
            buf = bufs.at[bslot]
            tbuf = tbufs.at[bslot]
            iota = lax.iota(jnp.int32, 16)
            rows = []
            cols = []
            for g8 in range(_L // 16):
                raw16 = raw_v[sq, pl.ds(jj * _L + g8 * 16, 16)]
                cols.append((raw16 & 1) << 6)
                rows.append(iota + g8 * 16)

            for f in range(d):
                for g8 in range(_L // 16):
                    val = plsc.load_gather(buf, [rows[g8], cols[g8] + f])
                    tbuf[f, pl.ds(g8 * 16, 16)] = val

        def s_start(k, bslot):
            sq, jj = split(k)
            pltpu.async_copy(tbufs.at[bslot],
                             out_hbm.at[sq, :, pl.ds(b0 + jj * _L, _L)],
                             ssem[bslot])

        def s_wait(bslot):
            pltpu.make_async_copy(tbufs.at[bslot],
                                  out_hbm.at[0, :, pl.ds(0, _L)],
                                  ssem[bslot]).wait()

        for bslot in range(_NBUF):
            g_start(bslot, bslot)

        def group(g, carry):
            for bslot in range(_NBUF):
                g_wait(bslot)
                merge(g * _NBUF + bslot, bslot)
                s_start(g * _NBUF + bslot, bslot)
            for bslot in range(_NBUF):
                s_wait(bslot)
                g_start((g + 1) * _NBUF + bslot, bslot)
            return carry

        lax.fori_loop(0, n_groups - 1, group, 0)

        last = (n_groups - 1) * _NBUF
        for bslot in range(_NBUF):
            g_wait(bslot)
            merge(last + bslot, bslot)
            s_start(last + bslot, bslot)
        for bslot in range(_NBUF):
            s_wait(bslot)

    return gather


def kernel(ids, table):
    b, s = ids.shape
    v, d = table.shape
    ids32 = ids.astype(jnp.int32)
    ids_t = ids32.T
    half_t = (ids32 >> 1).T
    table2 = table.reshape(v // 2, 2 * d)
    out5 = _make_gather(b, s, d)(ids_t, half_t, table2)
    return jnp.transpose(out5, (2, 0, 1))


# speedup vs baseline: 1.1977x; 1.1977x over previous
import functools

import jax
import jax.numpy as jnp
from jax import lax
from jax.experimental import pallas as pl
from jax.experimental.pallas import tpu as pltpu
from jax.experimental.pallas import tpu_sc as plsc

_L = 128
_NBUF = 2


def _make_gather(b: int, s: int, d: int):
    info = plsc.get_sparse_core_info()
    nc, ns = info.num_cores, info.num_subcores
    nw = nc * ns
    assert b % (nw * _L) == 0
    b_w = b // nw
    jmax = b_w // _L
    chunks_w = s * jmax
    assert chunks_w % _NBUF == 0
    n_groups = chunks_w // _NBUF
    mesh = plsc.VectorSubcoreMesh(core_axis_name="c", subcore_axis_name="s")

    @functools.partial(
        pl.kernel,
        mesh=mesh,
        out_type=jax.ShapeDtypeStruct((s, d, b), jnp.float32),
        scratch_types=[
            pltpu.VMEM((s, b_w), jnp.int32),
            pltpu.VMEM((s, b_w), jnp.int32),
            pltpu.VMEM((_NBUF, _L, _L), jnp.float32),
            pltpu.VMEM((_NBUF, d, _L), jnp.float32),
            [pltpu.SemaphoreType.DMA] * _NBUF,
            [pltpu.SemaphoreType.DMA] * _NBUF,
        ],
        compiler_params=pltpu.CompilerParams(needs_layout_passes=False),
    )
    def gather(ids_hbm, half_hbm, table_hbm, out_hbm, raw_v, half_v,
               bufs, tbufs, gsem, ssem):
        wid = lax.axis_index("s") * nc + lax.axis_index("c")
        b0 = wid * b_w

        pltpu.sync_copy(ids_hbm.at[:, pl.ds(b0, b_w)], raw_v)
        pltpu.sync_copy(half_hbm.at[:, pl.ds(b0, b_w)], half_v)

        def split(k):
            sq = k // jmax
            jj = k % jmax
            return sq, jj

        def g_start(k, bslot):
            sq, jj = split(k)
            pltpu.async_copy(
                table_hbm.at[half_v.at[sq, pl.ds(jj * _L, _L)]],
                bufs.at[bslot], gsem[bslot])

        def g_wait(bslot):
            pltpu.make_async_copy(table_hbm.at[half_v.at[0, pl.ds(0, _L)]],
                                  bufs.at[bslot], gsem[bslot]).wait()

        def merge(k, bslot):
            sq, jj = split(k)
            buf = bufs.at[bslot]
            tbuf = tbufs.at[bslot]
            iota = lax.iota(jnp.int32, 16)
            rows = []
            cols = []
            for g8 in range(_L // 16):
                raw16 = raw_v[sq, pl.ds(jj * _L + g8 * 16, 16)]
                rows.append(iota + g8 * 16)
                cols.append((raw16 & 1) << 6)

            ng = _L // 16
            for f in range(0, d, 2):
                vals = [plsc.load_gather(buf, [rows[g8], cols[g8] + f])
                        for g8 in range(ng)]
                vals += [plsc.load_gather(buf, [rows[g8], cols[g8] + (f + 1)])
                         for g8 in range(ng)]
                for g8 in range(ng):
                    tbuf[f, pl.ds(g8 * 16, 16)] = vals[g8]
                for g8 in range(ng):
                    tbuf[f + 1, pl.ds(g8 * 16, 16)] = vals[ng + g8]

        def s_start(k, bslot):
            sq, jj = split(k)
            pltpu.async_copy(tbufs.at[bslot],
                             out_hbm.at[sq, :, pl.ds(b0 + jj * _L, _L)],
                             ssem[bslot])

        def s_wait(bslot):
            pltpu.make_async_copy(tbufs.at[bslot],
                                  out_hbm.at[0, :, pl.ds(0, _L)],
                                  ssem[bslot]).wait()

        for bslot in range(_NBUF):
            g_start(bslot, bslot)

        def group(g, carry):
            for bslot in range(_NBUF):
                g_wait(bslot)
                merge(g * _NBUF + bslot, bslot)
                s_start(g * _NBUF + bslot, bslot)
            for bslot in range(_NBUF):
                s_wait(bslot)
                g_start((g + 1) * _NBUF + bslot, bslot)
            return carry

        lax.fori_loop(0, n_groups - 1, group, 0)

        last = (n_groups - 1) * _NBUF
        for bslot in range(_NBUF):
            g_wait(bslot)
            merge(last + bslot, bslot)
            s_start(last + bslot, bslot)
        for bslot in range(_NBUF):
            s_wait(bslot)

    return gather


def kernel(ids, table):
    b, s = ids.shape
    v, d = table.shape
    ids32 = ids.astype(jnp.int32)
    ids_t = ids32.T
    half_t = (ids32 >> 1).T
    table2 = table.reshape(v // 2, 2 * d)
    out5 = _make_gather(b, s, d)(ids_t, half_t, table2)
    return jnp.transpose(out5, (2, 0, 1))


# speedup vs baseline: 1.7376x; 1.4508x over previous
import functools

import jax
import jax.numpy as jnp
from jax import lax
from jax.experimental import pallas as pl
from jax.experimental.pallas import tpu as pltpu
from jax.experimental.pallas import tpu_sc as plsc

_L = 128
_NBUF = 2


def _make_gather(b: int, s: int, d: int):
    info = plsc.get_sparse_core_info()
    nc, ns = info.num_cores, info.num_subcores
    nw = nc * ns
    assert b % (nw * _L) == 0
    b_w = b // nw
    jmax = b_w // _L
    chunks_w = s * jmax
    assert chunks_w % _NBUF == 0
    n_groups = chunks_w // _NBUF
    mesh = plsc.VectorSubcoreMesh(core_axis_name="c", subcore_axis_name="s")

    @functools.partial(
        pl.kernel,
        mesh=mesh,
        out_type=jax.ShapeDtypeStruct((s, d, b), jnp.float32),
        scratch_types=[
            pltpu.VMEM((s, b_w), jnp.int32),
            pltpu.VMEM((s, b_w), jnp.int32),
            pltpu.VMEM((_NBUF, _L, _L), jnp.float32),
            pltpu.VMEM((_NBUF, d, _L), jnp.float32),
            [pltpu.SemaphoreType.DMA] * _NBUF,
            [pltpu.SemaphoreType.DMA] * _NBUF,
        ],
        compiler_params=pltpu.CompilerParams(needs_layout_passes=False),
    )
    def gather(ids_hbm, half_hbm, table_hbm, out_hbm, raw_v, half_v,
               bufs, tbufs, gsem, ssem):
        wid = lax.axis_index("s") * nc + lax.axis_index("c")
        b0 = wid * b_w

        pltpu.sync_copy(ids_hbm.at[:, pl.ds(b0, b_w)], raw_v)
        pltpu.sync_copy(half_hbm.at[:, pl.ds(b0, b_w)], half_v)

        def split(k):
            sq = k // jmax
            jj = k % jmax
            return sq, jj

        def g_start(k, bslot):
            sq, jj = split(k)
            pltpu.async_copy(
                table_hbm.at[half_v.at[sq, pl.ds(jj * _L, _L)]],
                bufs.at[bslot], gsem[bslot])

        def g_wait(bslot):
            pltpu.make_async_copy(table_hbm.at[half_v.at[0, pl.ds(0, _L)]],
                                  bufs.at[bslot], gsem[bslot]).wait()

        def merge(k, bslot):
            sq, jj = split(k)
            buf = bufs.at[bslot]
            tbuf = tbufs.at[bslot]
            iota = lax.iota(jnp.int32, 16)
            rot = [(iota + dd) & 15 for dd in range(16)]

            def jb_body(jb, carry):
                rowv = iota + jb * 16
                raw16 = raw_v[sq, pl.ds(jj * _L + jb * 16, 16)]
                pv = (raw16 & 1) << 6
                for fb in range(d // 16):
                    base = pv + fb * 16
                    for dd in range(16):
                        colv = base + rot[dd]
                        val = plsc.load_gather(buf, [rowv, colv])
                        fv = rot[dd] + fb * 16
                        plsc.store_scatter(tbuf, [fv, rowv], val)
                return carry

            lax.fori_loop(0, _L // 16, jb_body, 0)

        def s_start(k, bslot):
            sq, jj = split(k)
            pltpu.async_copy(tbufs.at[bslot],
                             out_hbm.at[sq, :, pl.ds(b0 + jj * _L, _L)],
                             ssem[bslot])

        def s_wait(bslot):
            pltpu.make_async_copy(tbufs.at[bslot],
                                  out_hbm.at[0, :, pl.ds(0, _L)],
                                  ssem[bslot]).wait()

        for bslot in range(_NBUF):
            g_start(bslot, bslot)

        def group(g, carry):
            for bslot in range(_NBUF):
                g_wait(bslot)
                merge(g * _NBUF + bslot, bslot)
                s_start(g * _NBUF + bslot, bslot)
            for bslot in range(_NBUF):
                s_wait(bslot)
                g_start((g + 1) * _NBUF + bslot, bslot)
            return carry

        lax.fori_loop(0, n_groups - 1, group, 0)

        last = (n_groups - 1) * _NBUF
        for bslot in range(_NBUF):
            g_wait(bslot)
            merge(last + bslot, bslot)
            s_start(last + bslot, bslot)
        for bslot in range(_NBUF):
            s_wait(bslot)

    return gather


def kernel(ids, table):
    b, s = ids.shape
    v, d = table.shape
    ids32 = ids.astype(jnp.int32)
    ids_t = ids32.T
    half_t = (ids32 >> 1).T
    table2 = table.reshape(v // 2, 2 * d)
    out5 = _make_gather(b, s, d)(ids_t, half_t, table2)
    return jnp.transpose(out5, (2, 0, 1))
